# Initial kernel scaffold; baseline (speedup 1.0000x reference)
import jax
import jax.numpy as jnp
from jax.experimental import pallas as pl


def kernel(token_ids, table):
    raise NotImplementedError("write your pallas kernel here")


# speedup vs baseline: 1.6836x; 1.6836x over previous
import jax
import jax.numpy as jnp
from jax import lax
from jax.experimental import pallas as pl
from jax.experimental.pallas import tpu as pltpu
from jax.experimental.pallas import tpu_sc as plsc

EMBED_DIM = 64
BATCH = 16384
HIST = 50
B_FLAT = BATCH * HIST

NUM_CORES = 2
NUM_SUBCORES = 16
NW = NUM_CORES * NUM_SUBCORES
PER_W = B_FLAT // NW
GATHER = 128
STEPS = PER_W // GATHER


def _emb_body(idx_hbm, table_hbm, out_hbm, idx_v, rows_v, sem):
    wid = lax.axis_index("s") * NUM_CORES + lax.axis_index("c")
    pltpu.sync_copy(idx_hbm.at[wid], idx_v)

    def step(j, carry):
        pltpu.async_copy(table_hbm.at[idx_v.at[j]], rows_v, sem).wait()
        pltpu.sync_copy(rows_v, out_hbm.at[wid, j])
        return carry

    lax.fori_loop(0, STEPS, step, 0)


def kernel(token_ids, table):
    idx = token_ids.astype(jnp.int32).reshape(NW, STEPS, GATHER)
    f = pl.kernel(
        _emb_body,
        out_type=jax.ShapeDtypeStruct((NW, STEPS, GATHER, EMBED_DIM),
                                      jnp.float32),
        mesh=plsc.VectorSubcoreMesh(core_axis_name="c", subcore_axis_name="s"),
        scratch_types=[
            pltpu.VMEM((STEPS, GATHER), jnp.int32),
            pltpu.VMEM((GATHER, EMBED_DIM), jnp.float32),
            pltpu.SemaphoreType.DMA,
        ],
        compiler_params=pltpu.CompilerParams(use_tc_tiling_on_sc=False),
    )
    out = f(idx, table)
    return out.reshape(BATCH, HIST, EMBED_DIM)


# speedup vs baseline: 1.8733x; 1.1127x over previous
import jax
import jax.numpy as jnp
from jax import lax
from jax.experimental import pallas as pl
from jax.experimental.pallas import tpu as pltpu
from jax.experimental.pallas import tpu_sc as plsc

EMBED_DIM = 64
BATCH = 16384
HIST = 50
B_FLAT = BATCH * HIST

NUM_CORES = 2
NUM_SUBCORES = 16
NW = NUM_CORES * NUM_SUBCORES
PER_W = B_FLAT // NW
GATHER = 128
STEPS = PER_W // GATHER
K = 5
GROUPS = STEPS // (2 * K)


def _emb_body(idx_hbm, table_hbm, out_hbm, idx_v, rows_v, gsem_a, gsem_b,
              ssem_a, ssem_b):
    wid = lax.axis_index("s") * NUM_CORES + lax.axis_index("c")
    pltpu.sync_copy(idx_hbm.at[wid], idx_v)

    def fire_gather(j, grp, b, sem):
        pltpu.make_async_copy(
            table_hbm.at[idx_v.at[j]], rows_v.at[grp, b], sem).start()

    def drain(j, grp, b, sem):
        pltpu.make_async_copy(
            table_hbm.at[idx_v.at[j]], rows_v.at[grp, b], sem).wait()

    def fire_store(j, grp, b, sem):
        pltpu.make_async_copy(
            rows_v.at[grp, b], out_hbm.at[wid, j], sem).start()

    def drain_store(j, grp, b, sem):
        pltpu.make_async_copy(
            rows_v.at[grp, b], out_hbm.at[wid, j], sem).wait()

    for b in range(K):
        fire_gather(b, 0, b, gsem_a)

    def outer(i, carry):
        base = i * 2 * K
        for b in range(K):
            drain(base + b, 0, b, gsem_a)
        for b in range(K):
            fire_store(base + b, 0, b, ssem_a)

        @pl.when(i > 0)
        def _():
            for b in range(K):
                drain_store(base - K + b, 1, b, ssem_b)

        for b in range(K):
            fire_gather(base + K + b, 1, b, gsem_b)
        for b in range(K):
            drain(base + K + b, 1, b, gsem_b)
        for b in range(K):
            fire_store(base + K + b, 1, b, ssem_b)

        for b in range(K):
            drain_store(base + b, 0, b, ssem_a)

        @pl.when(i + 1 < GROUPS)
        def _():
            for b in range(K):
                fire_gather(base + 2 * K + b, 0, b, gsem_a)
        return carry

    lax.fori_loop(0, GROUPS, outer, 0)

    last = (GROUPS - 1) * 2 * K + K
    for b in range(K):
        drain_store(last + b, 1, b, ssem_b)


def kernel(token_ids, table):
    idx = token_ids.astype(jnp.int32).reshape(NW, STEPS, GATHER)
    f = pl.kernel(
        _emb_body,
        out_type=jax.ShapeDtypeStruct((NW, STEPS, GATHER, EMBED_DIM),
                                      jnp.float32),
        mesh=plsc.VectorSubcoreMesh(core_axis_name="c", subcore_axis_name="s"),
        scratch_types=[
            pltpu.VMEM((STEPS, GATHER), jnp.int32),
            pltpu.VMEM((2, K, GATHER, EMBED_DIM), jnp.float32),
            pltpu.SemaphoreType.DMA,
            pltpu.SemaphoreType.DMA,
            pltpu.SemaphoreType.DMA,
            pltpu.SemaphoreType.DMA,
        ],
        compiler_params=pltpu.CompilerParams(use_tc_tiling_on_sc=False),
    )
    out = f(idx, table)
    return out.reshape(BATCH, HIST, EMBED_DIM)
